# baseline (speedup 1.0000x reference)
import functools

import jax
import jax.numpy as jnp
from jax import lax
from jax.experimental import pallas as pl
from jax.experimental.pallas import tpu as pltpu
from jax.experimental.pallas import tpu_sc as plsc

D = 64
S = 200
Q = S // 2
NC = 2
NS = 16
NW = NC * NS
BBLK = 128


def _body(idx_hbm, pos_hbm, table_hbm, out_hbm,
          pos_v, ib0, ib1, pa0, pa1, pb0, pb1, ha0, ha1, hb0, hb1,
          ga0, ga1, gb0, gb1, ob0, ob1, oi0, oi1,
          isem0, isem1, gsem0, gsem1, osem0, osem1):
    idxb = (ib0, ib1)
    pidxa = (pa0, pa1)
    pidxb = (pb0, pb1)
    h64a = (ha0, ha1)
    h64b = (hb0, hb1)
    gbufa = (ga0, ga1)
    gbufb = (gb0, gb1)
    obuf = (ob0, ob1)
    oidx = (oi0, oi1)
    isem = (isem0, isem1)
    gsem = (gsem0, gsem1)
    osem = (osem0, osem1)

    wid = lax.axis_index("sub") * NC + lax.axis_index("core")
    b0 = wid * BBLK

    pltpu.sync_copy(pos_hbm, pos_v)

    iota16 = jnp.arange(16, dtype=jnp.int32)
    zeros16 = jnp.zeros((16,), jnp.int32)
    c100 = iota16 * 100

    def idx_start(q, c):
        pltpu.async_copy(idx_hbm.at[pl.ds(2 * q, 2), pl.ds(b0, BBLK)],
                         idxb[c], isem[c])

    def idx_wait(q, c):
        pltpu.make_async_copy(idx_hbm.at[pl.ds(2 * q, 2), pl.ds(b0, BBLK)],
                              idxb[c], isem[c]).wait()

    def prep(c):
        for j in range(BBLK // 16):
            sl = pl.ds(16 * j, 16)
            va = idxb[c][0, sl]
            vb = idxb[c][1, sl]
            pidxa[c][sl] = lax.shift_right_logical(va, 1)
            pidxb[c][sl] = lax.shift_right_logical(vb, 1)
            h64a[c][sl] = lax.shift_left(jnp.bitwise_and(va, 1), 6)
            h64b[c][sl] = lax.shift_left(jnp.bitwise_and(vb, 1), 6)

    def gather_start(c):
        pltpu.async_copy(table_hbm.at[pidxa[c]], gbufa[c], gsem[c])
        pltpu.async_copy(table_hbm.at[pidxb[c]], gbufb[c], gsem[c])

    def gather_wait(c):
        pltpu.make_async_copy(table_hbm.at[pidxa[c]], gbufa[c], gsem[c]).wait()
        pltpu.make_async_copy(table_hbm.at[pidxb[c]], gbufb[c], gsem[c]).wait()

    def out_start(q, c):
        base = b0 * 100 + q
        for k in range(BBLK // 16):
            oidx[c][pl.ds(16 * k, 16)] = c100 + (base + 1600 * k)
        pltpu.async_copy(obuf[c], out_hbm.at[oidx[c]], osem[c])

    def out_wait(c):
        pltpu.make_async_copy(obuf[c], out_hbm.at[oidx[c]], osem[c]).wait()

    def build(q, c):
        ga = gbufa[c]
        gb = gbufb[c]
        ob = obuf[c]
        posv = [pos_v[q, pl.ds(16 * j, 16)] for j in range(8)]
        for k in range(BBLK // 16):
            rbase = zeros16 + 16 * k

            @plsc.parallel_loop(0, 16, unroll=4)
            def _(m):
                rv = rbase + jnp.bitwise_and(iota16 + m, 15)
                ha = plsc.load_gather(h64a[c], [rv])
                hb = plsc.load_gather(h64b[c], [rv])
                for j in range(4):
                    cseg = 16 * j + iota16
                    va = plsc.load_gather(ga, [rv, ha + cseg])
                    plsc.store_scatter(ob, [rv, cseg], va + posv[j])
                    vb = plsc.load_gather(gb, [rv, hb + cseg])
                    plsc.store_scatter(ob, [rv, 64 + cseg], vb + posv[4 + j])

    idx_start(0, 0)
    idx_start(1, 1)
    idx_wait(0, 0)
    prep(0)
    gather_start(0)

    def step(q, carry):
        c = lax.rem(q, 2)

        def qb(body, *args):
            @pl.when(c == 0)
            def _():
                body(*args, 0)

            @pl.when(c == 1)
            def _():
                body(*args, 1)

        qb(gather_wait)

        @pl.when(q + 1 < Q)
        def _():
            def nxt(cn):
                idx_wait(q + 1, cn)
                prep(cn)
                gather_start(cn)

            @pl.when(c == 0)
            def _():
                nxt(1)

            @pl.when(c == 1)
            def _():
                nxt(0)

        @pl.when(q + 2 < Q)
        def _():
            qb(idx_start, q + 2)

        @pl.when(q >= 2)
        def _():
            qb(out_wait)

        qb(build, q)
        qb(out_start, q)
        return carry

    lax.fori_loop(0, Q, step, 0)
    out_wait(0)
    out_wait(1)


def kernel(output, word_table, pos_table):
    batch, seq = output.shape
    vocab, d = word_table.shape
    assert batch == NW * BBLK and d == D and seq == S

    idx_t = output.T.astype(jnp.int32)
    pos_flat = pos_table.reshape(Q, 2 * D)
    wt_pairs = word_table.reshape(vocab // 2, 2 * D)

    mesh = plsc.VectorSubcoreMesh(core_axis_name="core", subcore_axis_name="sub")
    k = functools.partial(
        pl.kernel,
        mesh=mesh,
        out_type=jax.ShapeDtypeStruct((batch * Q, 2 * D), jnp.float32),
        scratch_types=[
            pltpu.VMEM((Q, 2 * D), jnp.float32),
            pltpu.VMEM((2, BBLK), jnp.int32),
            pltpu.VMEM((2, BBLK), jnp.int32),
            pltpu.VMEM((BBLK,), jnp.int32),
            pltpu.VMEM((BBLK,), jnp.int32),
            pltpu.VMEM((BBLK,), jnp.int32),
            pltpu.VMEM((BBLK,), jnp.int32),
            pltpu.VMEM((BBLK,), jnp.int32),
            pltpu.VMEM((BBLK,), jnp.int32),
            pltpu.VMEM((BBLK,), jnp.int32),
            pltpu.VMEM((BBLK,), jnp.int32),
            pltpu.VMEM((BBLK, 2 * D), jnp.float32),
            pltpu.VMEM((BBLK, 2 * D), jnp.float32),
            pltpu.VMEM((BBLK, 2 * D), jnp.float32),
            pltpu.VMEM((BBLK, 2 * D), jnp.float32),
            pltpu.VMEM((BBLK, 2 * D), jnp.float32),
            pltpu.VMEM((BBLK, 2 * D), jnp.float32),
            pltpu.VMEM((BBLK,), jnp.int32),
            pltpu.VMEM((BBLK,), jnp.int32),
            pltpu.SemaphoreType.DMA,
            pltpu.SemaphoreType.DMA,
            pltpu.SemaphoreType.DMA,
            pltpu.SemaphoreType.DMA,
            pltpu.SemaphoreType.DMA,
            pltpu.SemaphoreType.DMA,
        ],
        compiler_params=pltpu.CompilerParams(use_tc_tiling_on_sc=True,
                                             needs_layout_passes=False),
    )(_body)

    out_flat = k(idx_t, pos_flat, wt_pairs)
    return out_flat.reshape(batch, S, D)
